# baseline (speedup 1.0000x reference)
import jax
import jax.numpy as jnp
from jax import lax
from jax.experimental import pallas as pl
from jax.experimental.pallas import tpu as pltpu, tpu_sc as plsc

VOCAB = 1000000
EMBED_DIM = 64
BATCH = 4096
SEQ = 200
PADDED_DIM = 128

NC = 2
NS = 16
NW = NC * NS

BI = 128
NBT = BATCH // BI
CI = 8
NCT = EMBED_DIM // CI
N_UNITS = SEQ


def _make_gather():
    mesh = plsc.VectorSubcoreMesh(core_axis_name="c", subcore_axis_name="s")

    @pl.kernel(
        out_type=jax.ShapeDtypeStruct((SEQ, NCT, NBT, CI, BI), jnp.float32),
        mesh=mesh,
        scratch_types=[
            pltpu.VMEM((SEQ, BI), jnp.int32),
            [pltpu.VMEM((BI, PADDED_DIM), jnp.float32) for _ in range(4)],
            [pltpu.VMEM((EMBED_DIM, BI + 1), jnp.float32) for _ in range(4)],
            pltpu.SemaphoreType.DMA,
            [pltpu.SemaphoreType.DMA for _ in range(4)],
        ],
        compiler_params=pltpu.CompilerParams(
            use_tc_tiling_on_sc=False, needs_layout_passes=False),
    )
    def k(table_hbm, idx_hbm, out_hbm, idx_v, gbufs, tbufs, gsem, osems):
        wid = lax.axis_index("s") * NC + lax.axis_index("c")
        pltpu.sync_copy(idx_hbm.at[wid], idx_v)

        lane = lax.iota(jnp.int32, 16)

        def gather_descr(u, gbuf):
            return pltpu.make_async_copy(table_hbm.at[idx_v.at[u]], gbuf, gsem)

        def out_descrs(u, tbuf, osem):
            return [
                pltpu.make_async_copy(
                    tbuf.at[pl.ds(ct * CI, CI), pl.ds(0, BI)],
                    out_hbm.at[u, ct, wid], osem)
                for ct in range(NCT)
            ]

        def out_start(u, tbuf, osem):
            for d in out_descrs(u, tbuf, osem):
                d.start()

        def out_wait(u, tbuf, osem):
            for d in out_descrs(u, tbuf, osem):
                d.wait()

        cvecs = [c0 + lane for c0 in range(0, EMBED_DIM, 16)]
        klanes = [k * 16 + lane for k in range(EMBED_DIM // 16)]

        def transpose(gbuf, tbuf):
            def b_body(b, _):
                rowv = jnp.full((16,), b, jnp.int32)
                for k in range(EMBED_DIM // 16):
                    vals = plsc.load_gather(gbuf, [rowv, klanes[k]])
                    plsc.store_scatter(tbuf, [cvecs[k], rowv], vals)
                return ()
            lax.fori_loop(0, BI, b_body, (), unroll=8)

        NB = 4

        def step(u, j, do_wait_out, do_fire_next, last=False):
            gather_descr(u, gbufs[j]).wait()
            if do_wait_out:
                out_wait(u - NB, tbufs[j], osems[j])
            transpose(gbufs[j], tbufs[j])
            out_start(u, tbufs[j], osems[j])
            if do_fire_next:
                gather_descr(u + NB, gbufs[j]).start()

        for j in range(NB):
            gather_descr(j, gbufs[j]).start()
        for j in range(NB):
            step(j, j, False, True)

        def body(i, _):
            for j in range(NB):
                step(NB * i + j, j, True, True)
            return ()

        lax.fori_loop(1, N_UNITS // NB - 1, body, (), unroll=False)

        for j in range(NB):
            step(N_UNITS - NB + j, j, True, False)
        for j in range(NB):
            out_wait(N_UNITS - NB + j, tbufs[j], osems[j])

    return k


_gather = _make_gather()


def kernel(tokens, embed_weight):
    table = jnp.pad(embed_weight, ((0, 0), (0, PADDED_DIM - EMBED_DIM)))
    idx = tokens.astype(jnp.int32).T.reshape(SEQ, NBT, BI).transpose(1, 0, 2)
    out5 = _gather(table, idx)
    return out5.transpose(2, 4, 0, 1, 3).reshape(BATCH, SEQ, EMBED_DIM)


# speedup vs baseline: 1.3095x; 1.3095x over previous
import jax
import jax.numpy as jnp
from jax import lax
from jax.experimental import pallas as pl
from jax.experimental.pallas import tpu as pltpu, tpu_sc as plsc

VOCAB = 1000000
EMBED_DIM = 64
BATCH = 4096
SEQ = 200
PADDED_DIM = 128

NC = 2
NS = 16
NW = NC * NS

BI = 128
NBT = BATCH // BI
CI = 8
NCT = EMBED_DIM // CI
N_UNITS = SEQ


def _make_gather():
    mesh = plsc.VectorSubcoreMesh(core_axis_name="c", subcore_axis_name="s")

    @pl.kernel(
        out_type=jax.ShapeDtypeStruct((SEQ, NCT, NBT, CI, BI), jnp.float32),
        mesh=mesh,
        scratch_types=[
            pltpu.VMEM((SEQ, BI), jnp.int32),
            [pltpu.VMEM((BI, PADDED_DIM), jnp.float32) for _ in range(4)],
            [pltpu.VMEM((EMBED_DIM, BI + 1), jnp.float32) for _ in range(4)],
            pltpu.SemaphoreType.DMA,
            [pltpu.SemaphoreType.DMA for _ in range(4)],
        ],
        compiler_params=pltpu.CompilerParams(
            use_tc_tiling_on_sc=False, needs_layout_passes=False),
    )
    def k(table_hbm, idx_hbm, out_hbm, idx_v, gbufs, tbufs, gsem, osems):
        wid = lax.axis_index("s") * NC + lax.axis_index("c")
        pltpu.sync_copy(idx_hbm.at[wid], idx_v)

        lane = lax.iota(jnp.int32, 16)

        def gather_descr(u, gbuf):
            return pltpu.make_async_copy(table_hbm.at[idx_v.at[u]], gbuf, gsem)

        def out_descrs(u, tbuf, osem):
            return [
                pltpu.make_async_copy(
                    tbuf.at[pl.ds(ct * CI, CI), pl.ds(0, BI)],
                    out_hbm.at[u, ct, wid], osem)
                for ct in range(NCT)
            ]

        def out_start(u, tbuf, osem):
            for d in out_descrs(u, tbuf, osem):
                d.start()

        def out_wait(u, tbuf, osem):
            for d in out_descrs(u, tbuf, osem):
                d.wait()

        cvecs = [c0 + lane for c0 in range(0, EMBED_DIM, 16)]
        klanes = [k * 16 + lane for k in range(EMBED_DIM // 16)]

        def transpose(gbuf, tbuf):
            @plsc.parallel_loop(0, BI, unroll=8)
            def b_body(b):
                colv = jnp.full((16,), b, jnp.int32)
                for k in range(EMBED_DIM // 16):
                    vals = gbuf[b, pl.ds(k * 16, 16)]
                    plsc.store_scatter(tbuf, [cvecs[k], colv], vals)

        NB = 4

        def step(u, j, do_wait_out, do_fire_next, last=False):
            gather_descr(u, gbufs[j]).wait()
            if do_wait_out:
                out_wait(u - NB, tbufs[j], osems[j])
            transpose(gbufs[j], tbufs[j])
            out_start(u, tbufs[j], osems[j])
            if do_fire_next:
                gather_descr(u + NB, gbufs[j]).start()

        for j in range(NB):
            gather_descr(j, gbufs[j]).start()
        for j in range(NB):
            step(j, j, False, True)

        def body(i, _):
            for j in range(NB):
                step(NB * i + j, j, True, True)
            return ()

        lax.fori_loop(1, N_UNITS // NB - 1, body, (), unroll=False)

        for j in range(NB):
            step(N_UNITS - NB + j, j, True, False)
        for j in range(NB):
            out_wait(N_UNITS - NB + j, tbufs[j], osems[j])

    return k


_gather = _make_gather()


def kernel(tokens, embed_weight):
    table = jnp.pad(embed_weight, ((0, 0), (0, PADDED_DIM - EMBED_DIM)))
    idx = tokens.astype(jnp.int32).T.reshape(SEQ, NBT, BI).transpose(1, 0, 2)
    out5 = _gather(table, idx)
    return out5.transpose(2, 4, 0, 1, 3).reshape(BATCH, SEQ, EMBED_DIM)
